# Initial kernel scaffold; baseline (speedup 1.0000x reference)
import jax
import jax.numpy as jnp
from jax.experimental import pallas as pl


def kernel(feature, condition, edge_index, W_f, b_f, W_c, b_c, W_h, b_h, W_m, b_m, W_v, b_v):
    raise NotImplementedError("write your pallas kernel here")


# speedup vs baseline: 14.6449x; 14.6449x over previous
import functools

import jax
import jax.numpy as jnp
from jax import lax
from jax.experimental import pallas as pl
from jax.experimental.pallas import tpu as pltpu
from jax.experimental.pallas import tpu_sc as plsc

N = 10000
E = 320000
D = 128
NC, NS = 2, 16
NW = NC * NS
EPW = E // NW
C = 128
NFULL, TAIL = EPW // C, EPW % C
RPS = 624
_CHUNKS = [(0, 128), (128, 128), (256, 128), (384, 128), (512, 112)]
_TAIL_ROWS = (NS * RPS, N - NS * RPS)
DEGW = 16


def _zero_vmem(buf, nrows, width):
    @pl.loop(0, nrows)
    def _(i):
        for j in range(width // 16):
            buf[i, pl.ds(j * 16, 16)] = jnp.zeros((16,), jnp.float32)


def _deg_body(dst_hbm, deg_hbm, acc, dstbuf, dstbuf_t, ones, bounce, sem):
    cid = lax.axis_index("c")
    sid = lax.axis_index("s")
    wid = cid * NS + sid

    @pl.loop(0, C)
    def _(i):
        buf16 = jnp.ones((16,), jnp.float32)
        ones[i, pl.ds(0, 16)] = buf16
    _zero_vmem(bounce, C, DEGW)

    r0 = sid * RPS
    for off, m in _CHUNKS:
        pltpu.sync_copy(bounce.at[pl.ds(0, m)], acc.at[pl.ds(r0 + off, m)])

    @pl.when(sid == NS - 1)
    def _():
        t0, tm = _TAIL_ROWS
        pltpu.sync_copy(bounce.at[pl.ds(0, tm)], acc.at[pl.ds(t0, tm)])
    plsc.subcore_barrier()

    base_w = wid * EPW

    @pl.loop(0, NFULL)
    def _(i):
        base = pl.multiple_of(base_w + i * C, 8)
        pltpu.sync_copy(dst_hbm.at[pl.ds(base, C)], dstbuf)
        pltpu.sync_copy(ones, acc.at[dstbuf], add=True)

    base = pl.multiple_of(base_w + NFULL * C, 8)
    pltpu.sync_copy(dst_hbm.at[pl.ds(base, TAIL)], dstbuf_t)
    pltpu.sync_copy(ones.at[pl.ds(0, TAIL)], acc.at[dstbuf_t], add=True)
    plsc.subcore_barrier()

    for off, m in _CHUNKS:
        rr = r0 + off
        pltpu.sync_copy(acc.at[pl.ds(rr, m)], bounce.at[pl.ds(0, m)])
        pltpu.sync_copy(bounce.at[pl.ds(0, m)], deg_hbm.at[cid, pl.ds(rr, m)])

    @pl.when(sid == NS - 1)
    def _():
        t0, tm = _TAIL_ROWS
        pltpu.sync_copy(acc.at[pl.ds(t0, tm)], bounce.at[pl.ds(0, tm)])
        pltpu.sync_copy(bounce.at[pl.ds(0, tm)], deg_hbm.at[cid, pl.ds(t0, tm)])


def _prop_body(y_hbm, src_hbm, dst_hbm, p_hbm,
               acc, srcbuf, dstbuf, srcbuf_t, dstbuf_t, rows, sem):
    cid = lax.axis_index("c")
    sid = lax.axis_index("s")
    wid = cid * NS + sid

    _zero_vmem(rows, C, D)
    r0 = sid * RPS
    for off, m in _CHUNKS:
        pltpu.sync_copy(rows.at[pl.ds(0, m)], acc.at[pl.ds(r0 + off, m)])

    @pl.when(sid == NS - 1)
    def _():
        t0, tm = _TAIL_ROWS
        pltpu.sync_copy(rows.at[pl.ds(0, tm)], acc.at[pl.ds(t0, tm)])
    plsc.subcore_barrier()

    base_w = wid * EPW

    @pl.loop(0, NFULL)
    def _(i):
        base = pl.multiple_of(base_w + i * C, 8)
        pltpu.sync_copy(src_hbm.at[pl.ds(base, C)], srcbuf)
        pltpu.sync_copy(dst_hbm.at[pl.ds(base, C)], dstbuf)
        pltpu.async_copy(y_hbm.at[srcbuf], rows, sem).wait()
        pltpu.sync_copy(rows, acc.at[dstbuf], add=True)

    base = pl.multiple_of(base_w + NFULL * C, 8)
    pltpu.sync_copy(src_hbm.at[pl.ds(base, TAIL)], srcbuf_t)
    pltpu.sync_copy(dst_hbm.at[pl.ds(base, TAIL)], dstbuf_t)
    pltpu.async_copy(y_hbm.at[srcbuf_t], rows.at[pl.ds(0, TAIL)], sem).wait()
    pltpu.sync_copy(rows.at[pl.ds(0, TAIL)], acc.at[dstbuf_t], add=True)
    plsc.subcore_barrier()

    for off, m in _CHUNKS:
        rr = r0 + off
        pltpu.sync_copy(acc.at[pl.ds(rr, m)], rows.at[pl.ds(0, m)])
        pltpu.sync_copy(rows.at[pl.ds(0, m)], p_hbm.at[cid, pl.ds(rr, m)])

    @pl.when(sid == NS - 1)
    def _():
        t0, tm = _TAIL_ROWS
        pltpu.sync_copy(acc.at[pl.ds(t0, tm)], rows.at[pl.ds(0, tm)])
        pltpu.sync_copy(rows.at[pl.ds(0, tm)], p_hbm.at[cid, pl.ds(t0, tm)])


@functools.cache
def _sc_mesh():
    return plsc.VectorSubcoreMesh(core_axis_name="c", subcore_axis_name="s",
                                  num_cores=NC, num_subcores=NS)


@jax.jit
def _sc_degree(dst):
    return pl.kernel(
        _deg_body,
        out_type=jax.ShapeDtypeStruct((NC, N, DEGW), jnp.float32),
        mesh=_sc_mesh(),
        scratch_types=[
            pltpu.VMEM_SHARED((N, DEGW), jnp.float32),
            pltpu.VMEM((C,), jnp.int32),
            pltpu.VMEM((TAIL,), jnp.int32),
            pltpu.VMEM((C, DEGW), jnp.float32),
            pltpu.VMEM((C, DEGW), jnp.float32),
            pltpu.SemaphoreType.DMA,
        ],
    )(dst)


@jax.jit
def _sc_propagate(y, src, dst):
    return pl.kernel(
        _prop_body,
        out_type=jax.ShapeDtypeStruct((NC, N, D), jnp.float32),
        mesh=_sc_mesh(),
        scratch_types=[
            pltpu.VMEM_SHARED((N, D), jnp.float32),
            pltpu.VMEM((C,), jnp.int32),
            pltpu.VMEM((C,), jnp.int32),
            pltpu.VMEM((TAIL,), jnp.int32),
            pltpu.VMEM((TAIL,), jnp.int32),
            pltpu.VMEM((C, D), jnp.float32),
            pltpu.SemaphoreType.DMA,
        ],
    )(y, src, dst)


_RB = 2000
_GRID = (N // _RB,)


def _tc_call(body, out_shapes, in_specs, out_specs):
    return pl.pallas_call(
        body,
        grid=_GRID,
        in_specs=in_specs,
        out_specs=out_specs,
        out_shape=out_shapes,
    )


def _rows(w):
    return pl.BlockSpec((_RB, w), lambda i: (i, 0))


def _part(w):
    return pl.BlockSpec((NC, _RB, w), lambda i: (0, i, 0))


def _full(a, b):
    return pl.BlockSpec((a, b), lambda i: (0, 0))


def _k1_body(degp, f, c, wf, wc, dis_o, yf_o, yc_o):
    deg = degp[0, :, 0:1] + degp[1, :, 0:1] + 1.0
    dis = lax.rsqrt(deg)
    dis_o[...] = dis
    yf_o[...] = dis * jnp.dot(f[...], wf[...],
                              preferred_element_type=jnp.float32)
    yc_o[...] = dis * jnp.dot(c[...], wc[...],
                              preferred_element_type=jnp.float32)


def _k2_body(pf, yf, bf, pc, yc, bc, dis, wh1, wh2, yh_o):
    d = dis[...]
    f2h = jnp.tanh(d * (pf[0] + pf[1] + yf[...]) + bf[...])
    c2h = jnp.tanh(d * (pc[0] + pc[1] + yc[...]) + bc[...])
    yh_o[...] = d * (jnp.dot(f2h, wh1[...], preferred_element_type=jnp.float32)
                     + jnp.dot(c2h, wh2[...], preferred_element_type=jnp.float32))


def _k3_body(ph, yh, bh, dis, y2_o):
    d = dis[...]
    h2 = jnp.tanh(d * (ph[0] + ph[1] + yh[...]) + bh[...])
    y2_o[...] = d * h2


def _k4_body(p2, y2, dis, wm, bm, wv, bv, noise, z_o, mean_o, logvar_o):
    ah = dis[...] * (p2[0] + p2[1] + y2[...])
    mean = jnp.dot(ah, wm[...], preferred_element_type=jnp.float32) + bm[...]
    logvar = jnp.dot(ah, wv[...], preferred_element_type=jnp.float32) + bv[...]
    mean_o[...] = mean
    logvar_o[...] = logvar
    z_o[...] = noise[...] * jnp.exp(0.5 * logvar) + mean


def kernel(feature, condition, edge_index, W_f, b_f, W_c, b_c, W_h, b_h,
           W_m, b_m, W_v, b_v):
    src = edge_index[0].astype(jnp.int32)
    dst = edge_index[1].astype(jnp.int32)
    cdim = condition.shape[1]
    ldim = W_m.shape[1]

    degp = _sc_degree(dst)

    k1 = _tc_call(
        _k1_body,
        (jax.ShapeDtypeStruct((N, 1), jnp.float32),
         jax.ShapeDtypeStruct((N, D), jnp.float32),
         jax.ShapeDtypeStruct((N, D), jnp.float32)),
        [_part(DEGW), _rows(D), _rows(cdim), _full(D, D), _full(cdim, D)],
        (_rows(1), _rows(D), _rows(D)),
    )
    dis, yf, yc = k1(degp, feature, condition, W_f, W_c)

    pf = _sc_propagate(yf, src, dst)
    pc = _sc_propagate(yc, src, dst)

    b2 = lambda b: b.reshape(1, -1)
    k2 = _tc_call(
        _k2_body,
        jax.ShapeDtypeStruct((N, D), jnp.float32),
        [_part(D), _rows(D), _full(1, D), _part(D), _rows(D), _full(1, D),
         _rows(1), _full(D, D), _full(D, D)],
        _rows(D),
    )
    yh = k2(pf, yf, b2(b_f), pc, yc, b2(b_c), dis, W_h[:D], W_h[D:])

    ph = _sc_propagate(yh, src, dst)

    k3 = _tc_call(
        _k3_body,
        jax.ShapeDtypeStruct((N, D), jnp.float32),
        [_part(D), _rows(D), _full(1, D), _rows(1)],
        _rows(D),
    )
    y2 = k3(ph, yh, b2(b_h), dis)

    p2 = _sc_propagate(y2, src, dst)

    noise = jax.random.normal(jax.random.key(42), (N, ldim), jnp.float32)
    k4 = _tc_call(
        _k4_body,
        (jax.ShapeDtypeStruct((N, ldim), jnp.float32),
         jax.ShapeDtypeStruct((N, ldim), jnp.float32),
         jax.ShapeDtypeStruct((N, ldim), jnp.float32)),
        [_part(D), _rows(D), _rows(1), _full(D, ldim), _full(1, ldim),
         _full(D, ldim), _full(1, ldim), _rows(ldim)],
        (_rows(ldim), _rows(ldim), _rows(ldim)),
    )
    z, mean, logvar = k4(p2, y2, dis, W_m, b2(b_m), W_v, b2(b_v), noise)
    return (z, mean, logvar)
